# Initial kernel scaffold; baseline (speedup 1.0000x reference)
import functools

import jax
import jax.numpy as jnp
from jax import lax
from jax.experimental import pallas as pl
from jax.experimental.pallas import tpu as pltpu
from jax.experimental.pallas import tpu_sc as plsc

N_TOT = 16384
NCLS = 128
DF = 128
NSUP = 10
NQ2 = 20
BLK = 512
NBLK = N_TOT // BLK
NSP = N_TOT + 128
INF = float("inf")


def _prep_kernel(y_ref, sortpos_ref, off_ref, cnt_ref, carry, offs, carry2):
    p = pl.program_id(0)
    b = pl.program_id(1)
    y = y_ref[0]
    cls_iota = lax.broadcasted_iota(jnp.int32, (NCLS, BLK), 0)
    oh = (cls_iota == y).astype(jnp.float32)

    @pl.when((p == 0) & (b == 0))
    def _init():
        carry[...] = jnp.zeros_like(carry)

    @pl.when(p == 0)
    def _phase0():
        carry[...] = carry[...] + jnp.sum(oh, axis=1, keepdims=True)

    @pl.when((p == 0) & (b == NBLK - 1))
    def _mkoff():
        cnt = carry[...]
        r = lax.broadcasted_iota(jnp.int32, (NCLS, NCLS), 0)
        cc = lax.broadcasted_iota(jnp.int32, (NCLS, NCLS), 1)
        lst = (cc < r).astype(jnp.float32)
        off = jnp.dot(lst, cnt, preferred_element_type=jnp.float32)
        offs[...] = off
        carry2[...] = jnp.zeros_like(carry2)
        off_ref[...] = off.astype(jnp.int32)
        cnt_ref[...] = cnt.astype(jnp.int32)

    @pl.when(p == 1)
    def _phase1():
        r = lax.broadcasted_iota(jnp.int32, (BLK, BLK), 0)
        cc = lax.broadcasted_iota(jnp.int32, (BLK, BLK), 1)
        tri = (r <= cc).astype(jnp.float32)
        cums = jnp.dot(oh, tri, preferred_element_type=jnp.float32)
        pos = oh * (offs[...] + carry2[...] - 1.0 + cums)
        sortpos_ref[0] = jnp.sum(pos, axis=0, keepdims=True).astype(jnp.int32)
        carry2[...] = carry2[...] + jnp.sum(oh, axis=1, keepdims=True)


def _make_prep(interpret=False):
    return pl.pallas_call(
        _prep_kernel,
        grid=(2, NBLK),
        in_specs=[pl.BlockSpec((1, 1, BLK), lambda p, b: (b, 0, 0))],
        out_specs=[
            pl.BlockSpec((1, 1, BLK), lambda p, b: (b, 0, 0)),
            pl.BlockSpec((NCLS, 1), lambda p, b: (0, 0)),
            pl.BlockSpec((NCLS, 1), lambda p, b: (0, 0)),
        ],
        out_shape=[
            jax.ShapeDtypeStruct((NBLK, 1, BLK), jnp.int32),
            jax.ShapeDtypeStruct((NCLS, 1), jnp.int32),
            jax.ShapeDtypeStruct((NCLS, 1), jnp.int32),
        ],
        scratch_shapes=[
            pltpu.VMEM((NCLS, 1), jnp.float32),
            pltpu.VMEM((NCLS, 1), jnp.float32),
            pltpu.VMEM((NCLS, 1), jnp.float32),
        ],
        compiler_params=pltpu.CompilerParams(
            dimension_semantics=("arbitrary", "arbitrary")),
        interpret=interpret,
    )


def _sc_scatter_body(x_hbm, pos_hbm, out_hbm, pos_v, rows_v, sem):
    cid = lax.axis_index("c")
    sid = lax.axis_index("s")
    wid = sid * 2 + cid
    pltpu.sync_copy(pos_hbm.at[wid], pos_v)
    for j in range(4):
        pltpu.sync_copy(x_hbm.at[pl.ds(wid * 512 + j * 128, 128)], rows_v)
        pltpu.async_copy(rows_v, out_hbm.at[pos_v.at[j]], sem).wait()


def _make_scatter():
    mesh = plsc.VectorSubcoreMesh(core_axis_name="c", subcore_axis_name="s")
    return functools.partial(
        pl.kernel,
        mesh=mesh,
        out_type=jax.ShapeDtypeStruct((NSP, DF), jnp.float32),
        scratch_types=[
            pltpu.VMEM((4, 128), jnp.int32),
            pltpu.VMEM((128, DF), jnp.float32),
            pltpu.SemaphoreType.DMA,
        ],
    )(_sc_scatter_body)


def _dist_rows(qT, ptv):
    rows = []
    for i in range(128):
        qcol = qT[:, i:i + 1]
        d = qcol - ptv
        sq = d * d
        s = sq[0:8]
        for k in range(1, 16):
            s = s + sq[8 * k:8 * k + 8]
        r4 = s[0:4] + s[4:8]
        r2 = r4[0:2] + r4[2:4]
        rows.append(r2[0:1] + r2[1:2])
    return rows


def _select_kernel(off_s, cnt_s, xs_ref, loss_ref, accn_ref, yhat_ref,
                   pT, dS):
    c = pl.program_id(0)

    @pl.when(c == 0)
    def _setup():
        loss_ref[...] = jnp.zeros_like(loss_ref)
        accn_ref[...] = jnp.zeros_like(accn_ref)

        def build(j, _):
            oj = off_s[j, 0]
            a = xs_ref[pl.ds(oj, 8), :]
            bfull = xs_ref[pl.ds(oj + 8, 8), :]
            sl = lax.broadcasted_iota(jnp.int32, (8, DF), 0)
            bm = jnp.where(sl < NSUP - 8, bfull, 0.0)
            t = a + bm
            r4 = t[0:4] + t[4:8]
            r2 = r4[0:2] + r4[2:4]
            r1 = r2[0:1] + r2[1:2]
            dS[pl.ds(j, 1), :] = r1 / 10.0
            return 0

        lax.fori_loop(0, NCLS, build, 0)
        pT[...] = jnp.transpose(dS[...])

    qs = off_s[c, 0] + NSUP
    nq = cnt_s[c, 0] - NSUP
    ntiles = lax.max((nq + 127) // 128, 0)
    ptv = pT[...]

    def tile_body(t, acc):
        base = qs + t * 128
        q = xs_ref[pl.ds(base, 128), :]
        qT = jnp.transpose(q)
        rows = _dist_rows(qT, ptv)
        for i in range(128):
            dS[pl.ds(i, 1), :] = rows[i]
        rio = lax.broadcasted_iota(jnp.int32, (128, NCLS), 0)
        dm = jnp.where(rio < (nq - t * 128), dS[...], INF)
        w = jnp.concatenate([dm, acc], axis=0)
        rio2 = lax.broadcasted_iota(jnp.int32, (160, NCLS), 0)
        picked = []
        for _ in range(NQ2):
            m = jnp.min(w, axis=0, keepdims=True)
            eq = w == m
            fi = jnp.min(jnp.where(eq, rio2, 999), axis=0, keepdims=True)
            w = jnp.where(eq & (rio2 == fi), INF, w)
            picked.append(m)
        pad = jnp.full((32 - NQ2, NCLS), INF, jnp.float32)
        return jnp.concatenate(picked + [pad], axis=0)

    acc0 = jnp.full((32, NCLS), INF, jnp.float32)
    acc = lax.fori_loop(0, ntiles, tile_body, acc0)

    acc_pad = jnp.concatenate(
        [acc, jnp.full((128 - 32, NCLS), INF, jnp.float32)], axis=0)
    acc_t = jnp.transpose(acc_pad)
    neg = -acc_t
    mx = jnp.max(neg, axis=0, keepdims=True)
    e = jnp.exp(neg - mx)
    ssum = jnp.sum(e, axis=0, keepdims=True)
    lse = mx + jnp.log(ssum)

    rio3 = lax.broadcasted_iota(jnp.int32, (NCLS, 128), 0)
    val = jnp.sum(jnp.where(rio3 == c, acc_t, 0.0), axis=0, keepdims=True)
    lane = lax.broadcasted_iota(jnp.int32, (1, 128), 1)
    contrib = jnp.where(lane < NQ2, val + lse, 0.0)
    loss_ref[...] = loss_ref[...] + contrib

    dmin = jnp.min(acc_t, axis=0, keepdims=True)
    eqy = acc_t == dmin
    idx = jnp.min(jnp.where(eqy, rio3, NCLS), axis=0, keepdims=True)
    yhat_ref[...] = idx
    correct = jnp.where((lane < NQ2) & (idx == c), 1.0, 0.0)
    accn_ref[...] = accn_ref[...] + correct


def _make_select(interpret=False):
    return pl.pallas_call(
        _select_kernel,
        grid=(NCLS,),
        in_specs=[
            pl.BlockSpec(memory_space=pltpu.SMEM),
            pl.BlockSpec(memory_space=pltpu.SMEM),
            pl.BlockSpec((NSP, DF), lambda c: (0, 0)),
        ],
        out_specs=[
            pl.BlockSpec((1, 128), lambda c: (0, 0)),
            pl.BlockSpec((1, 128), lambda c: (0, 0)),
            pl.BlockSpec((1, 128), lambda c: (c, 0)),
        ],
        out_shape=[
            jax.ShapeDtypeStruct((1, 128), jnp.float32),
            jax.ShapeDtypeStruct((1, 128), jnp.float32),
            jax.ShapeDtypeStruct((NCLS, 128), jnp.int32),
        ],
        scratch_shapes=[
            pltpu.VMEM((DF, NCLS), jnp.float32),
            pltpu.VMEM((128, NCLS), jnp.float32),
        ],
        compiler_params=pltpu.CompilerParams(
            dimension_semantics=("arbitrary",)),
        interpret=interpret,
    )


def kernel(input, target):
    x = input
    y = target.astype(jnp.int32).reshape(NBLK, 1, BLK)
    sortpos, off, cnt = _make_prep()(y)
    xs = _make_scatter()(x, sortpos.reshape(32, 4, 128))
    loss_vec, accn_vec, yhat = _make_select()(off, cnt, xs)
    loss = jnp.sum(loss_vec) / 2560.0
    accv = jnp.sum(accn_vec) / 2560.0
    return (loss, accv, yhat[:, :NQ2])


# speedup vs baseline: 182.6776x; 182.6776x over previous
import functools

import jax
import jax.numpy as jnp
from jax import lax
from jax.experimental import pallas as pl
from jax.experimental.pallas import tpu as pltpu
from jax.experimental.pallas import tpu_sc as plsc

N_TOT = 16384
NCLS = 128
DF = 128
NSUP = 10
NQ2 = 20
BLK = 512
NBLK = N_TOT // BLK
NSP = N_TOT + 128
INF = float("inf")


def _prep_kernel(y_ref, sortpos_ref, off_ref, cnt_ref, carry, offs, carry2):
    p = pl.program_id(0)
    b = pl.program_id(1)
    y = y_ref[0]
    cls_iota = lax.broadcasted_iota(jnp.int32, (NCLS, BLK), 0)
    oh = (cls_iota == y).astype(jnp.float32)

    @pl.when((p == 0) & (b == 0))
    def _init():
        carry[...] = jnp.zeros_like(carry)

    @pl.when(p == 0)
    def _phase0():
        carry[...] = carry[...] + jnp.sum(oh, axis=1, keepdims=True)

    @pl.when((p == 0) & (b == NBLK - 1))
    def _mkoff():
        cnt = carry[...]
        r = lax.broadcasted_iota(jnp.int32, (NCLS, NCLS), 0)
        cc = lax.broadcasted_iota(jnp.int32, (NCLS, NCLS), 1)
        lst = (cc < r).astype(jnp.float32)
        off = jnp.dot(lst, cnt, preferred_element_type=jnp.float32)
        offs[...] = off
        carry2[...] = jnp.zeros_like(carry2)
        off_ref[...] = off.astype(jnp.int32)
        cnt_ref[...] = cnt.astype(jnp.int32)

    @pl.when(p == 1)
    def _phase1():
        r = lax.broadcasted_iota(jnp.int32, (BLK, BLK), 0)
        cc = lax.broadcasted_iota(jnp.int32, (BLK, BLK), 1)
        tri = (r <= cc).astype(jnp.float32)
        cums = jnp.dot(oh, tri, preferred_element_type=jnp.float32)
        pos = oh * (offs[...] + carry2[...] - 1.0 + cums)
        sortpos_ref[0] = jnp.sum(pos, axis=0, keepdims=True).astype(jnp.int32)
        carry2[...] = carry2[...] + jnp.sum(oh, axis=1, keepdims=True)


def _make_prep(interpret=False):
    return pl.pallas_call(
        _prep_kernel,
        grid=(2, NBLK),
        in_specs=[pl.BlockSpec((1, 1, BLK), lambda p, b: (b, 0, 0))],
        out_specs=[
            pl.BlockSpec((1, 1, BLK), lambda p, b: (b, 0, 0)),
            pl.BlockSpec((NCLS, 1), lambda p, b: (0, 0)),
            pl.BlockSpec((NCLS, 1), lambda p, b: (0, 0)),
        ],
        out_shape=[
            jax.ShapeDtypeStruct((NBLK, 1, BLK), jnp.int32),
            jax.ShapeDtypeStruct((NCLS, 1), jnp.int32),
            jax.ShapeDtypeStruct((NCLS, 1), jnp.int32),
        ],
        scratch_shapes=[
            pltpu.VMEM((NCLS, 1), jnp.float32),
            pltpu.VMEM((NCLS, 1), jnp.float32),
            pltpu.VMEM((NCLS, 1), jnp.float32),
        ],
        compiler_params=pltpu.CompilerParams(
            dimension_semantics=("arbitrary", "arbitrary")),
        interpret=interpret,
    )


def _sc_scatter_body(x_hbm, pos_hbm, out_hbm, pos_v, rows_v, sem):
    cid = lax.axis_index("c")
    sid = lax.axis_index("s")
    wid = sid * 2 + cid
    pltpu.sync_copy(pos_hbm.at[wid], pos_v)
    for j in range(4):
        pltpu.sync_copy(x_hbm.at[pl.ds(wid * 512 + j * 128, 128)], rows_v)
        pltpu.async_copy(rows_v, out_hbm.at[pos_v.at[j]], sem).wait()


def _make_scatter():
    mesh = plsc.VectorSubcoreMesh(core_axis_name="c", subcore_axis_name="s")
    return functools.partial(
        pl.kernel,
        mesh=mesh,
        out_type=jax.ShapeDtypeStruct((NSP, DF), jnp.float32),
        scratch_types=[
            pltpu.VMEM((4, 128), jnp.int32),
            pltpu.VMEM((128, DF), jnp.float32),
            pltpu.SemaphoreType.DMA,
        ],
    )(_sc_scatter_body)


def _dist_rows(qT, ptv):
    rows = []
    for i in range(128):
        qcol = qT[:, i:i + 1]
        d = qcol - ptv
        sq = d * d
        s = sq[0:8]
        for k in range(1, 16):
            s = s + sq[8 * k:8 * k + 8]
        r4 = s[0:4] + s[4:8]
        r2 = r4[0:2] + r4[2:4]
        rows.append(r2[0:1] + r2[1:2])
    return rows


def _select_kernel(off_s, cnt_s, xs_ref, loss_ref, accn_ref, yhat_ref,
                   pT, dS):
    c = pl.program_id(0)

    @pl.when(c == 0)
    def _setup():
        loss_ref[...] = jnp.zeros_like(loss_ref)
        accn_ref[...] = jnp.zeros_like(accn_ref)

        def build(j, _):
            oj = off_s[j, 0]
            a = xs_ref[pl.ds(oj, 8), :]
            bfull = xs_ref[pl.ds(oj + 8, 8), :]
            sl = lax.broadcasted_iota(jnp.int32, (8, DF), 0)
            bm = jnp.where(sl < NSUP - 8, bfull, 0.0)
            t = a + bm
            r4 = t[0:4] + t[4:8]
            r2 = r4[0:2] + r4[2:4]
            r1 = r2[0:1] + r2[1:2]
            dS[pl.ds(j, 1), :] = r1 / 10.0
            return 0

        lax.fori_loop(0, NCLS, build, 0)
        pT[...] = jnp.transpose(dS[...])

    qs = off_s[c, 0] + NSUP
    nq = cnt_s[c, 0] - NSUP
    ntiles = lax.max((nq + 127) // 128, 0)
    ptv = pT[...]

    def tile_body(t, acc):
        base = qs + t * 128
        q = xs_ref[pl.ds(base, 128), :]
        qT = jnp.transpose(q)
        rows = _dist_rows(qT, ptv)
        for i in range(128):
            dS[pl.ds(i, 1), :] = rows[i]
        rio = lax.broadcasted_iota(jnp.int32, (128, NCLS), 0)
        dm = jnp.where(rio < (nq - t * 128), dS[...], INF)
        w = jnp.concatenate([dm, acc], axis=0)
        rio2 = lax.broadcasted_iota(jnp.int32, (160, NCLS), 0)
        picked = []
        for _ in range(NQ2):
            m = jnp.min(w, axis=0, keepdims=True)
            eq = w == m
            fi = jnp.min(jnp.where(eq, rio2, 999), axis=0, keepdims=True)
            w = jnp.where(eq & (rio2 == fi), INF, w)
            picked.append(m)
        pad = jnp.full((32 - NQ2, NCLS), INF, jnp.float32)
        return jnp.concatenate(picked + [pad], axis=0)

    acc0 = jnp.full((32, NCLS), INF, jnp.float32)
    acc = lax.fori_loop(0, ntiles, tile_body, acc0)

    acc_pad = jnp.concatenate(
        [acc, jnp.full((128 - 32, NCLS), INF, jnp.float32)], axis=0)
    acc_t = jnp.transpose(acc_pad)
    neg = -acc_t
    mx = jnp.max(neg, axis=0, keepdims=True)
    e = jnp.exp(neg - mx)
    ssum = jnp.sum(e, axis=0, keepdims=True)
    lse = mx + jnp.log(ssum)

    rio3 = lax.broadcasted_iota(jnp.int32, (NCLS, 128), 0)
    val = jnp.sum(jnp.where(rio3 == c, acc_t, 0.0), axis=0, keepdims=True)
    lane = lax.broadcasted_iota(jnp.int32, (1, 128), 1)
    contrib = jnp.where(lane < NQ2, val + lse, 0.0)
    loss_ref[...] = loss_ref[...] + contrib

    dmin = jnp.min(acc_t, axis=0, keepdims=True)
    eqy = acc_t == dmin
    idx = jnp.min(jnp.where(eqy, rio3, NCLS), axis=0, keepdims=True)
    yhat_ref[0] = idx
    correct = jnp.where((lane < NQ2) & (idx == c), 1.0, 0.0)
    accn_ref[...] = accn_ref[...] + correct


def _make_select(interpret=False):
    return pl.pallas_call(
        _select_kernel,
        grid=(NCLS,),
        in_specs=[
            pl.BlockSpec(memory_space=pltpu.SMEM),
            pl.BlockSpec(memory_space=pltpu.SMEM),
            pl.BlockSpec((NSP, DF), lambda c: (0, 0)),
        ],
        out_specs=[
            pl.BlockSpec((1, 128), lambda c: (0, 0)),
            pl.BlockSpec((1, 128), lambda c: (0, 0)),
            pl.BlockSpec((1, 1, 128), lambda c: (c, 0, 0)),
        ],
        out_shape=[
            jax.ShapeDtypeStruct((1, 128), jnp.float32),
            jax.ShapeDtypeStruct((1, 128), jnp.float32),
            jax.ShapeDtypeStruct((NCLS, 1, 128), jnp.int32),
        ],
        scratch_shapes=[
            pltpu.VMEM((DF, NCLS), jnp.float32),
            pltpu.VMEM((128, NCLS), jnp.float32),
        ],
        compiler_params=pltpu.CompilerParams(
            dimension_semantics=("arbitrary",)),
        interpret=interpret,
    )


def kernel(input, target):
    x = input
    y = target.astype(jnp.int32).reshape(NBLK, 1, BLK)
    sortpos, off, cnt = _make_prep()(y)
    xs = _make_scatter()(x, sortpos.reshape(32, 4, 128))
    loss_vec, accn_vec, yhat = _make_select()(off, cnt, xs)
    loss = jnp.sum(loss_vec) / 2560.0
    accv = jnp.sum(accn_vec) / 2560.0
    return (loss, accv, yhat.reshape(NCLS, 128)[:, :NQ2])
